# baseline (speedup 1.0000x reference)
import jax
import jax.numpy as jnp
from jax import lax
from jax.experimental import pallas as pl
from jax.experimental.pallas import tpu as pltpu
from jax.experimental.pallas import tpu_sc as plsc

EMB = 16
NUM_FIELDS = 6
CAT = NUM_FIELDS * EMB
PADC = 128
_NC, _NS = 2, 16
_NW = _NC * _NS
_CH = 256


def _sc_gather_body(*refs):
    idx_refs = refs[0:6]
    tab_refs = refs[6:12]
    comb_hbm = refs[12]
    tabs_v = refs[13:19]
    idx_v = refs[19]
    comb_v = refs[20]
    sem_i = (refs[21], refs[22])
    sem_o = (refs[23], refs[24])

    n = idx_refs[0].shape[0]
    per_worker = n // _NW
    steps = per_worker // _CH

    wid = lax.axis_index("s") * _NC + lax.axis_index("c")
    w0 = wid * per_worker

    for f in range(NUM_FIELDS):
        pltpu.sync_copy(tab_refs[f], tabs_v[f])

    zeros16 = jnp.zeros((16,), jnp.float32)

    def zstep(r, _):
        for k in range(PADC // 16):
            comb_v[r, pl.ds(k * 16, 16)] = zeros16
        return ()

    lax.fori_loop(0, 2 * _CH, zstep, (), unroll=4)

    def fire_idx(s, buf):
        base = w0 + s * _CH
        for f in range(NUM_FIELDS):
            pltpu.async_copy(
                idx_refs[f].at[pl.ds(base, _CH)],
                idx_v.at[pl.ds((buf * NUM_FIELDS + f) * _CH, _CH)],
                sem_i[buf],
            )

    def wait_idx(buf):
        for f in range(NUM_FIELDS):
            pltpu.make_async_copy(
                idx_refs[f].at[pl.ds(0, _CH)],
                idx_v.at[pl.ds((buf * NUM_FIELDS + f) * _CH, _CH)],
                sem_i[buf],
            ).wait()

    def wait_out(buf):
        pltpu.make_async_copy(
            comb_v.at[pl.ds(buf * _CH, _CH), :],
            comb_hbm.at[pl.ds(0, _CH), :],
            sem_o[buf],
        ).wait()

    lanes = lax.iota(jnp.int32, 16)

    def substep(s, buf):
        wait_idx(buf)

        @pl.when(s + 1 < steps)
        def _():
            fire_idx(s + 1, 1 - buf)

        @pl.when(s >= 2)
        def _():
            wait_out(buf)

        @plsc.parallel_loop(0, _CH // 16, 1, unroll=2)
        def per_group(g):
            rvec = buf * _CH + g * 16 + lanes
            for f in range(NUM_FIELDS):
                ioff = (buf * NUM_FIELDS + f) * _CH + g * 16
                idx16 = idx_v[pl.ds(ioff, 16)]
                rbase = idx16 * EMB
                for j in range(EMB):
                    vals = plsc.load_gather(tabs_v[f], [rbase + j])
                    cvec = jnp.full((16,), f * EMB + j, jnp.int32)
                    plsc.store_scatter(comb_v, [rvec, cvec], vals)

        base = w0 + s * _CH
        pltpu.async_copy(
            comb_v.at[pl.ds(buf * _CH, _CH), :],
            comb_hbm.at[pl.ds(base, _CH), :],
            sem_o[buf],
        )

    fire_idx(0, 0)

    def pair(k, _):
        substep(2 * k, 0)
        substep(2 * k + 1, 1)
        return ()

    lax.fori_loop(0, steps // 2, pair, (), unroll=False)
    wait_out(0)
    wait_out(1)


def _sc_gather6(idx_list, tab_list):
    n = idx_list[0].shape[0]
    mesh = plsc.VectorSubcoreMesh(core_axis_name="c", subcore_axis_name="s")
    f = pl.kernel(
        _sc_gather_body,
        out_type=jax.ShapeDtypeStruct((n, PADC), jnp.float32),
        mesh=mesh,
        scratch_types=[pltpu.VMEM((t.size,), jnp.float32) for t in tab_list]
        + [
            pltpu.VMEM((2 * NUM_FIELDS * _CH,), jnp.int32),
            pltpu.VMEM((2 * _CH, PADC), jnp.float32),
            pltpu.SemaphoreType.DMA,
            pltpu.SemaphoreType.DMA,
            pltpu.SemaphoreType.DMA,
            pltpu.SemaphoreType.DMA,
        ],
        compiler_params=pltpu.CompilerParams(
            use_tc_tiling_on_sc=False, needs_layout_passes=False
        ),
    )
    return f(*idx_list, *[t.reshape(-1) for t in tab_list])


def _tc_fuse_body(comb_ref, w_ref, b_ref, g_ref, bt_ref, out_ref):
    W = w_ref[...]
    scale = jnp.clip(jnp.mean(jnp.abs(W)), 1e-5, None)
    Wq = jnp.clip(jnp.round(W / scale), -1.0, 1.0) * scale
    Wq = jnp.concatenate([Wq, jnp.zeros((W.shape[0], PADC - CAT), W.dtype)], axis=1)
    z = lax.dot_general(
        comb_ref[...], Wq, (((1,), (1,)), ((), ())),
        preferred_element_type=jnp.float32,
    )
    z = z + b_ref[...]
    mu = jnp.mean(z, axis=-1, keepdims=True)
    var = jnp.mean((z - mu) ** 2, axis=-1, keepdims=True)
    out_ref[...] = (z - mu) * lax.rsqrt(var + 1e-5) * g_ref[...] + bt_ref[...]


def _tc_fuse(comb, W, b, gamma, beta, tb=2048):
    n = comb.shape[0]
    d = W.shape[0]
    grid = (n // tb,)
    p_spec = pl.BlockSpec(W.shape, lambda i: (0, 0))
    v_spec = pl.BlockSpec((1, d), lambda i: (0, 0))
    return pl.pallas_call(
        _tc_fuse_body,
        grid=grid,
        in_specs=[pl.BlockSpec((tb, PADC), lambda i: (i, 0)), p_spec, v_spec,
                  v_spec, v_spec],
        out_specs=pl.BlockSpec((tb, d), lambda i: (i, 0)),
        out_shape=jax.ShapeDtypeStruct((n, d), jnp.float32),
    )(comb, W, b.reshape(1, d), gamma.reshape(1, d), beta.reshape(1, d))


def kernel(event_type, fault_class, syscall_class, opcode_family, transition_type,
           result_class, T_et, T_fc, T_sc, T_of, T_tt, T_rc, W, b, gamma, beta):
    bsz, seq = event_type.shape
    idx_list = [
        x.reshape(-1)
        for x in (event_type, fault_class, syscall_class, opcode_family,
                  transition_type, result_class)
    ]
    tab_list = [T_et, T_fc, T_sc, T_of, T_tt, T_rc]
    n = idx_list[0].shape[0]
    comb = _sc_gather6(idx_list, tab_list)
    out = _tc_fuse(comb, W, b, gamma, beta)
    return out.reshape(bsz, seq, W.shape[0])


# speedup vs baseline: 1.7171x; 1.7171x over previous
import jax
import jax.numpy as jnp
from jax import lax
from jax.experimental import pallas as pl
from jax.experimental.pallas import tpu as pltpu
from jax.experimental.pallas import tpu_sc as plsc

EMB = 16
NUM_FIELDS = 6
CAT = NUM_FIELDS * EMB
PADC = 128
_NC, _NS = 2, 16
_NW = _NC * _NS
_CH = 256


def _sc_gather_body(*refs):
    idx_refs = refs[0:6]
    tab_refs = refs[6:12]
    comb_hbm = refs[12]
    tabs_v = refs[13:19]
    idx_v = refs[19]
    comb_v = refs[20]
    sem_i = (refs[21], refs[22])
    sem_o = (refs[23], refs[24])

    n = idx_refs[0].shape[0]
    per_worker = n // _NW
    steps = per_worker // _CH

    wid = lax.axis_index("s") * _NC + lax.axis_index("c")
    w0 = wid * per_worker

    for f in range(NUM_FIELDS):
        pltpu.sync_copy(tab_refs[f], tabs_v[f])

    zeros16 = jnp.zeros((16,), jnp.float32)

    def zstep(r, _):
        for k in range(PADC // 16):
            comb_v[r, pl.ds(k * 16, 16)] = zeros16
        return ()

    lax.fori_loop(0, 2 * _CH, zstep, (), unroll=4)

    def fire_idx(s, buf):
        base = w0 + s * _CH
        for f in range(NUM_FIELDS):
            pltpu.async_copy(
                idx_refs[f].at[pl.ds(base, _CH)],
                idx_v.at[pl.ds((buf * NUM_FIELDS + f) * _CH, _CH)],
                sem_i[buf],
            )

    def wait_idx(buf):
        for f in range(NUM_FIELDS):
            pltpu.make_async_copy(
                idx_refs[f].at[pl.ds(0, _CH)],
                idx_v.at[pl.ds((buf * NUM_FIELDS + f) * _CH, _CH)],
                sem_i[buf],
            ).wait()

    def wait_out(buf):
        pltpu.make_async_copy(
            comb_v.at[pl.ds(buf * _CH, _CH), :],
            comb_hbm.at[pl.ds(0, _CH), :],
            sem_o[buf],
        ).wait()

    lanes = lax.iota(jnp.int32, 16)

    def substep(s, buf):
        wait_idx(buf)

        @pl.when(s + 1 < steps)
        def _():
            fire_idx(s + 1, 1 - buf)

        @pl.when(s >= 2)
        def _():
            wait_out(buf)

        @plsc.parallel_loop(0, _CH // 16, 1, unroll=2)
        def per_group(g):
            row0 = buf * _CH + g * 16
            for f in range(NUM_FIELDS):
                ioff = (buf * NUM_FIELDS + f) * _CH + g * 16
                idx16 = idx_v[pl.ds(ioff, 16)]
                for t in range(16):
                    addr = idx16[t] * EMB + lanes
                    vals = plsc.load_gather(tabs_v[f], [addr])
                    comb_v[row0 + t, pl.ds(f * EMB, EMB)] = vals

        base = w0 + s * _CH
        pltpu.async_copy(
            comb_v.at[pl.ds(buf * _CH, _CH), :],
            comb_hbm.at[pl.ds(base, _CH), :],
            sem_o[buf],
        )

    fire_idx(0, 0)

    def pair(k, _):
        substep(2 * k, 0)
        substep(2 * k + 1, 1)
        return ()

    lax.fori_loop(0, steps // 2, pair, (), unroll=False)
    wait_out(0)
    wait_out(1)


def _sc_gather6(idx_list, tab_list):
    n = idx_list[0].shape[0]
    mesh = plsc.VectorSubcoreMesh(core_axis_name="c", subcore_axis_name="s")
    f = pl.kernel(
        _sc_gather_body,
        out_type=jax.ShapeDtypeStruct((n, PADC), jnp.float32),
        mesh=mesh,
        scratch_types=[pltpu.VMEM((t.size,), jnp.float32) for t in tab_list]
        + [
            pltpu.VMEM((2 * NUM_FIELDS * _CH,), jnp.int32),
            pltpu.VMEM((2 * _CH, PADC), jnp.float32),
            pltpu.SemaphoreType.DMA,
            pltpu.SemaphoreType.DMA,
            pltpu.SemaphoreType.DMA,
            pltpu.SemaphoreType.DMA,
        ],
        compiler_params=pltpu.CompilerParams(
            use_tc_tiling_on_sc=False, needs_layout_passes=False
        ),
    )
    return f(*idx_list, *[t.reshape(-1) for t in tab_list])


def _tc_fuse_body(comb_ref, w_ref, b_ref, g_ref, bt_ref, out_ref):
    W = w_ref[...]
    scale = jnp.clip(jnp.mean(jnp.abs(W)), 1e-5, None)
    Wq = jnp.clip(jnp.round(W / scale), -1.0, 1.0) * scale
    Wq = jnp.concatenate([Wq, jnp.zeros((W.shape[0], PADC - CAT), W.dtype)], axis=1)
    z = lax.dot_general(
        comb_ref[...], Wq, (((1,), (1,)), ((), ())),
        preferred_element_type=jnp.float32,
    )
    z = z + b_ref[...]
    mu = jnp.mean(z, axis=-1, keepdims=True)
    var = jnp.mean((z - mu) ** 2, axis=-1, keepdims=True)
    out_ref[...] = (z - mu) * lax.rsqrt(var + 1e-5) * g_ref[...] + bt_ref[...]


def _tc_fuse(comb, W, b, gamma, beta, tb=2048):
    n = comb.shape[0]
    d = W.shape[0]
    grid = (n // tb,)
    p_spec = pl.BlockSpec(W.shape, lambda i: (0, 0))
    v_spec = pl.BlockSpec((1, d), lambda i: (0, 0))
    return pl.pallas_call(
        _tc_fuse_body,
        grid=grid,
        in_specs=[pl.BlockSpec((tb, PADC), lambda i: (i, 0)), p_spec, v_spec,
                  v_spec, v_spec],
        out_specs=pl.BlockSpec((tb, d), lambda i: (i, 0)),
        out_shape=jax.ShapeDtypeStruct((n, d), jnp.float32),
    )(comb, W, b.reshape(1, d), gamma.reshape(1, d), beta.reshape(1, d))


def kernel(event_type, fault_class, syscall_class, opcode_family, transition_type,
           result_class, T_et, T_fc, T_sc, T_of, T_tt, T_rc, W, b, gamma, beta):
    bsz, seq = event_type.shape
    idx_list = [
        x.reshape(-1)
        for x in (event_type, fault_class, syscall_class, opcode_family,
                  transition_type, result_class)
    ]
    tab_list = [T_et, T_fc, T_sc, T_of, T_tt, T_rc]
    n = idx_list[0].shape[0]
    comb = _sc_gather6(idx_list, tab_list)
    out = _tc_fuse(comb, W, b, gamma, beta)
    return out.reshape(bsz, seq, W.shape[0])


# speedup vs baseline: 2.1923x; 1.2767x over previous
import jax
import jax.numpy as jnp
from jax import lax
from jax.experimental import pallas as pl
from jax.experimental.pallas import tpu as pltpu
from jax.experimental.pallas import tpu_sc as plsc

EMB = 16
NUM_FIELDS = 6
CAT = NUM_FIELDS * EMB
PADC = 128
_NC, _NS = 2, 16
_NW = _NC * _NS
_CH = 256


def _sc_gather_body(*refs):
    idx_refs = refs[0:6]
    tab_refs = refs[6:12]
    comb_hbm = refs[12]
    tabs_v = refs[13:19]
    idx_v = refs[19]
    comb_v = refs[20]
    sem_i = (refs[21], refs[22])
    sem_o = (refs[23], refs[24])

    n = idx_refs[0].shape[0]
    per_worker = n // _NW
    steps = per_worker // _CH

    wid = lax.axis_index("s") * _NC + lax.axis_index("c")
    w0 = wid * per_worker

    for f in range(NUM_FIELDS):
        pltpu.sync_copy(tab_refs[f], tabs_v[f])

    zeros16 = jnp.zeros((16,), jnp.float32)

    def zstep(r, _):
        for k in range(PADC // 16):
            comb_v[r, pl.ds(k * 16, 16)] = zeros16
        return ()

    lax.fori_loop(0, 2 * _CH, zstep, (), unroll=4)

    def fire_idx(s, buf):
        base = w0 + s * _CH
        for f in range(NUM_FIELDS):
            pltpu.async_copy(
                idx_refs[f].at[pl.ds(base, _CH)],
                idx_v.at[pl.ds((buf * NUM_FIELDS + f) * _CH, _CH)],
                sem_i[buf],
            )

    def wait_idx(buf):
        for f in range(NUM_FIELDS):
            pltpu.make_async_copy(
                idx_refs[f].at[pl.ds(0, _CH)],
                idx_v.at[pl.ds((buf * NUM_FIELDS + f) * _CH, _CH)],
                sem_i[buf],
            ).wait()

    def wait_out(buf):
        pltpu.make_async_copy(
            comb_v.at[pl.ds(buf * _CH, _CH), :],
            comb_hbm.at[pl.ds(0, _CH), :],
            sem_o[buf],
        ).wait()

    lanes = lax.iota(jnp.int32, 16)

    def substep(s, buf):
        wait_idx(buf)

        @pl.when(s + 1 < steps)
        def _():
            fire_idx(s + 1, 1 - buf)

        @pl.when(s >= 2)
        def _():
            wait_out(buf)

        @plsc.parallel_loop(0, _CH // 16, 1, unroll=2)
        def per_group(g):
            row0 = buf * _CH + g * 16
            for f in range(NUM_FIELDS):
                ioff = (buf * NUM_FIELDS + f) * _CH + g * 16
                idx16 = idx_v[pl.ds(ioff, 16)]
                for t in range(16):
                    addr = idx16[t] * EMB + lanes
                    vals = plsc.load_gather(tabs_v[f], [addr])
                    comb_v[row0 + t, pl.ds(f * EMB, EMB)] = vals

        base = w0 + s * _CH
        pltpu.async_copy(
            comb_v.at[pl.ds(buf * _CH, _CH), :],
            comb_hbm.at[pl.ds(base, _CH), :],
            sem_o[buf],
        )

    fire_idx(0, 0)

    def pair(k, _):
        substep(2 * k, 0)
        substep(2 * k + 1, 1)
        return ()

    lax.fori_loop(0, steps // 2, pair, (), unroll=False)
    wait_out(0)
    wait_out(1)


def _sc_gather6(idx_list, tab_list):
    n = idx_list[0].shape[0]
    mesh = plsc.VectorSubcoreMesh(core_axis_name="c", subcore_axis_name="s")
    f = pl.kernel(
        _sc_gather_body,
        out_type=jax.ShapeDtypeStruct((n, PADC), jnp.float32),
        mesh=mesh,
        scratch_types=[pltpu.VMEM((t.size,), jnp.float32) for t in tab_list]
        + [
            pltpu.VMEM((2 * NUM_FIELDS * _CH,), jnp.int32),
            pltpu.VMEM((2 * _CH, PADC), jnp.float32),
            pltpu.SemaphoreType.DMA,
            pltpu.SemaphoreType.DMA,
            pltpu.SemaphoreType.DMA,
            pltpu.SemaphoreType.DMA,
        ],
        compiler_params=pltpu.CompilerParams(
            use_tc_tiling_on_sc=False, needs_layout_passes=False
        ),
    )
    return f(*idx_list, *[t.reshape(-1) for t in tab_list])


def _tc_fuse_body(comb_ref, w_ref, b_ref, g_ref, bt_ref, out_ref, *, rb, seq, d):
    W = w_ref[...]
    scale = jnp.clip(jnp.mean(jnp.abs(W)), 1e-5, None)
    Wq = jnp.clip(jnp.round(W / scale), -1.0, 1.0) * scale
    Wq = jnp.concatenate([Wq, jnp.zeros((W.shape[0], PADC - CAT), W.dtype)], axis=1)
    z = lax.dot_general(
        comb_ref[...], Wq, (((1,), (1,)), ((), ())),
        preferred_element_type=jnp.float32,
    )
    z = z + b_ref[...]
    mu = jnp.mean(z, axis=-1, keepdims=True)
    var = jnp.mean((z - mu) ** 2, axis=-1, keepdims=True)
    res = (z - mu) * lax.rsqrt(var + 1e-5) * g_ref[...] + bt_ref[...]
    out_ref[...] = res.reshape(out_ref.shape)


def _tc_fuse(comb, W, b, gamma, beta, bsz, seq, rb=32):
    d = W.shape[0]
    grid = (bsz // rb,)
    p_spec = pl.BlockSpec(W.shape, lambda i: (0, 0))
    v_spec = pl.BlockSpec((1, d), lambda i: (0, 0))
    return pl.pallas_call(
        lambda *a: _tc_fuse_body(*a, rb=rb, seq=seq, d=d),
        grid=grid,
        in_specs=[pl.BlockSpec((rb * seq, PADC), lambda i: (i, 0)), p_spec,
                  v_spec, v_spec, v_spec],
        out_specs=pl.BlockSpec((rb, seq, d), lambda i: (i, 0, 0)),
        out_shape=jax.ShapeDtypeStruct((bsz, seq, d), jnp.float32),
    )(comb, W, b.reshape(1, d), gamma.reshape(1, d), beta.reshape(1, d))


def kernel(event_type, fault_class, syscall_class, opcode_family, transition_type,
           result_class, T_et, T_fc, T_sc, T_of, T_tt, T_rc, W, b, gamma, beta):
    bsz, seq = event_type.shape
    idx_list = [
        x.reshape(-1)
        for x in (event_type, fault_class, syscall_class, opcode_family,
                  transition_type, result_class)
    ]
    tab_list = [T_et, T_fc, T_sc, T_of, T_tt, T_rc]
    n = idx_list[0].shape[0]
    comb = _sc_gather6(idx_list, tab_list)
    return _tc_fuse(comb, W, b, gamma, beta, bsz, seq)


# speedup vs baseline: 2.5298x; 1.1539x over previous
import jax
import jax.numpy as jnp
from jax import lax
from jax.experimental import pallas as pl
from jax.experimental.pallas import tpu as pltpu
from jax.experimental.pallas import tpu_sc as plsc

EMB = 16
NUM_FIELDS = 6
CAT = NUM_FIELDS * EMB
PADC = 128
_NC, _NS = 2, 16
_NW = _NC * _NS
_CH = 256


def _sc_gather_body(*refs):
    idx_refs = refs[0:6]
    tab_refs = refs[6:12]
    comb_hbm = refs[12]
    tabs_v = refs[13:19]
    idx_v = refs[19]
    comb_v = refs[20]
    sem_i = (refs[21], refs[22])
    sem_o = (refs[23], refs[24])

    n = idx_refs[0].shape[0]
    per_worker = n // _NW
    steps = per_worker // _CH

    wid = lax.axis_index("s") * _NC + lax.axis_index("c")
    w0 = wid * per_worker

    for f in range(NUM_FIELDS):
        pltpu.sync_copy(tab_refs[f], tabs_v[f])

    zeros16 = jnp.zeros((16,), jnp.float32)

    def zstep(r, _):
        for k in range(PADC // 16):
            comb_v[r, pl.ds(k * 16, 16)] = zeros16
        return ()

    lax.fori_loop(0, 2 * _CH, zstep, (), unroll=4)

    def fire_idx(s, buf):
        base = w0 + s * _CH
        for f in range(NUM_FIELDS):
            pltpu.async_copy(
                idx_refs[f].at[pl.ds(base, _CH)],
                idx_v.at[pl.ds((buf * NUM_FIELDS + f) * _CH, _CH)],
                sem_i[buf],
            )

    def wait_idx(buf):
        for f in range(NUM_FIELDS):
            pltpu.make_async_copy(
                idx_refs[f].at[pl.ds(0, _CH)],
                idx_v.at[pl.ds((buf * NUM_FIELDS + f) * _CH, _CH)],
                sem_i[buf],
            ).wait()

    def wait_out(buf):
        pltpu.make_async_copy(
            comb_v.at[pl.ds(buf * _CH, _CH), :],
            comb_hbm.at[pl.ds(0, _CH), :],
            sem_o[buf],
        ).wait()

    lanes = lax.iota(jnp.int32, 16)

    def substep(s, buf):
        wait_idx(buf)

        @pl.when(s + 1 < steps)
        def _():
            fire_idx(s + 1, 1 - buf)

        @pl.when(s >= 2)
        def _():
            wait_out(buf)

        @plsc.parallel_loop(0, _CH // 16, 1, unroll=2)
        def per_group(g):
            row0 = buf * _CH + g * 16
            for f in range(NUM_FIELDS):
                ioff = (buf * NUM_FIELDS + f) * _CH + g * 16
                idx16 = idx_v[pl.ds(ioff, 16)]
                for t in range(16):
                    addr = idx16[t] * EMB + lanes
                    vals = plsc.load_gather(tabs_v[f], [addr])
                    comb_v[row0 + t, pl.ds(f * EMB, EMB)] = vals

        base = w0 + s * _CH
        pltpu.async_copy(
            comb_v.at[pl.ds(buf * _CH, _CH), :],
            comb_hbm.at[pl.ds(base, _CH), :],
            sem_o[buf],
        )

    fire_idx(0, 0)

    def pair(k, _):
        substep(2 * k, 0)
        substep(2 * k + 1, 1)
        return ()

    lax.fori_loop(0, steps // 2, pair, (), unroll=False)
    wait_out(0)
    wait_out(1)


def _sc_gather6(idx_list, tab_list):
    n = idx_list[0].shape[0]
    mesh = plsc.VectorSubcoreMesh(core_axis_name="c", subcore_axis_name="s")
    f = pl.kernel(
        _sc_gather_body,
        out_type=jax.ShapeDtypeStruct((n, PADC), jnp.float32),
        mesh=mesh,
        scratch_types=[pltpu.VMEM((t.size,), jnp.float32) for t in tab_list]
        + [
            pltpu.VMEM((2 * NUM_FIELDS * _CH,), jnp.int32),
            pltpu.VMEM((2 * _CH, PADC), jnp.float32),
            pltpu.SemaphoreType.DMA,
            pltpu.SemaphoreType.DMA,
            pltpu.SemaphoreType.DMA,
            pltpu.SemaphoreType.DMA,
        ],
        compiler_params=pltpu.CompilerParams(
            use_tc_tiling_on_sc=False, needs_layout_passes=False
        ),
    )
    return f(*idx_list, *[t.reshape(-1) for t in tab_list])


def _tc_fuse_body(comb_ref, w_ref, b_ref, g_ref, bt_ref, out_ref, *, rb, seq, d):
    W = w_ref[...]
    scale = jnp.clip(jnp.mean(jnp.abs(W)), 1e-5, None)
    Wq = jnp.clip(jnp.round(W / scale), -1.0, 1.0) * scale
    Wq = jnp.concatenate([Wq, jnp.zeros((W.shape[0], PADC - CAT), W.dtype)], axis=1)
    z = lax.dot_general(
        comb_ref[...], Wq, (((1,), (1,)), ((), ())),
        preferred_element_type=jnp.float32,
    )
    z = z + b_ref[...]
    mu = jnp.mean(z, axis=-1, keepdims=True)
    var = jnp.mean((z - mu) ** 2, axis=-1, keepdims=True)
    res = (z - mu) * lax.rsqrt(var + 1e-5) * g_ref[...] + bt_ref[...]
    out_ref[...] = res.reshape(out_ref.shape)


def _tc_fuse(comb, W, b, gamma, beta, bsz, seq, rb=64):
    d = W.shape[0]
    grid = (bsz // rb,)
    p_spec = pl.BlockSpec(W.shape, lambda i: (0, 0))
    v_spec = pl.BlockSpec((1, d), lambda i: (0, 0))
    return pl.pallas_call(
        lambda *a: _tc_fuse_body(*a, rb=rb, seq=seq, d=d),
        grid=grid,
        in_specs=[pl.BlockSpec((rb * seq, PADC), lambda i: (i, 0)), p_spec,
                  v_spec, v_spec, v_spec],
        out_specs=pl.BlockSpec((rb, seq, d), lambda i: (i, 0, 0)),
        out_shape=jax.ShapeDtypeStruct((bsz, seq, d), jnp.float32),
    )(comb, W, b.reshape(1, d), gamma.reshape(1, d), beta.reshape(1, d))


def kernel(event_type, fault_class, syscall_class, opcode_family, transition_type,
           result_class, T_et, T_fc, T_sc, T_of, T_tt, T_rc, W, b, gamma, beta):
    bsz, seq = event_type.shape
    idx_list = [
        x.reshape(-1)
        for x in (event_type, fault_class, syscall_class, opcode_family,
                  transition_type, result_class)
    ]
    tab_list = [T_et, T_fc, T_sc, T_of, T_tt, T_rc]
    n = idx_list[0].shape[0]
    comb = _sc_gather6(idx_list, tab_list)
    return _tc_fuse(comb, W, b, gamma, beta, bsz, seq)
